# baseline (speedup 1.0000x reference)
import functools

import jax
import jax.numpy as jnp
from jax import lax
from jax.experimental import pallas as pl
from jax.experimental.pallas import tpu as pltpu
from jax.experimental.pallas import tpu_sc as plsc

SEQ_LEN = 4096
EMB = 1024
LANES = 16


def kernel(input, encoding):
    mesh = plsc.VectorSubcoreMesh(core_axis_name="c", subcore_axis_name="s")
    info = plsc.get_sparse_core_info()
    num_workers = info.num_cores * info.num_subcores
    rows_per_worker = SEQ_LEN // num_workers

    @functools.partial(
        pl.kernel,
        mesh=mesh,
        out_type=jax.ShapeDtypeStruct((SEQ_LEN, EMB), jnp.float32),
        scratch_types=[pltpu.VMEM((LANES,), jnp.int32)],
    )
    def body(inp_hbm, enc_hbm, out_hbm, inp_v):
        wid = lax.axis_index("s") * info.num_cores + lax.axis_index("c")
        pltpu.sync_copy(inp_hbm, inp_v.at[pl.ds(0, 2)])
        seq_from_input = inp_v[...][1]
        start = seq_from_input - SEQ_LEN
        base = wid * rows_per_worker
        src_row = pl.multiple_of(start + base, 8)
        dst_row = pl.multiple_of(base, 8)
        pltpu.sync_copy(
            enc_hbm.at[pl.ds(src_row, rows_per_worker)],
            out_hbm.at[pl.ds(dst_row, rows_per_worker)],
        )

    return body(input, encoding)


# speedup vs baseline: 16.5141x; 16.5141x over previous
import functools

import jax
import jax.numpy as jnp
from jax import lax
from jax.experimental import pallas as pl
from jax.experimental.pallas import tpu as pltpu
from jax.experimental.pallas import tpu_sc as plsc

SEQ_LEN = 4096
EMB = 1024
LANES = 16
CHUNK = 32


def kernel(input, encoding):
    mesh = plsc.VectorSubcoreMesh(core_axis_name="c", subcore_axis_name="s")
    info = plsc.get_sparse_core_info()
    num_workers = info.num_cores * info.num_subcores
    rows_per_worker = SEQ_LEN // num_workers
    nchunks = rows_per_worker // CHUNK

    @functools.partial(
        pl.kernel,
        mesh=mesh,
        out_type=jax.ShapeDtypeStruct((SEQ_LEN, EMB), jnp.float32),
        scratch_types=[
            pltpu.VMEM((LANES,), jnp.int32),
            pltpu.VMEM((2, CHUNK, EMB), jnp.float32),
            pltpu.SemaphoreType.DMA,
            pltpu.SemaphoreType.DMA,
            pltpu.SemaphoreType.DMA,
            pltpu.SemaphoreType.DMA,
        ],
    )
    def body(inp_hbm, enc_hbm, out_hbm, inp_v, buf, gs0, gs1, ss0, ss1):
        wid = lax.axis_index("s") * info.num_cores + lax.axis_index("c")
        pltpu.sync_copy(inp_hbm, inp_v.at[pl.ds(0, 2)])
        seq_from_input = inp_v[...][1]
        start = seq_from_input - SEQ_LEN
        base = wid * rows_per_worker
        src0 = pl.multiple_of(start + base, 8)
        dst0 = pl.multiple_of(base, 8)
        gsem = (gs0, gs1)
        ssem = (ss0, ss1)

        pend_g = [None, None]
        pend_s = [None, None]
        for i in range(nchunks):
            b = i & 1
            if pend_s[b] is not None:
                pend_s[b].wait()
            pend_g[b] = pltpu.async_copy(
                enc_hbm.at[pl.ds(src0 + i * CHUNK, CHUNK)], buf.at[b], gsem[b]
            )
            if i >= 1:
                ob = (i - 1) & 1
                pend_g[ob].wait()
                pend_s[ob] = pltpu.async_copy(
                    buf.at[ob], out_hbm.at[pl.ds(dst0 + (i - 1) * CHUNK, CHUNK)],
                    ssem[ob],
                )
        lb = (nchunks - 1) & 1
        pend_g[lb].wait()
        pend_s[lb] = pltpu.async_copy(
            buf.at[lb], out_hbm.at[pl.ds(dst0 + (nchunks - 1) * CHUNK, CHUNK)],
            ssem[lb],
        )
        pend_s[1 - lb].wait()
        pend_s[lb].wait()

    return body(input, encoding)


# speedup vs baseline: 16.5220x; 1.0005x over previous
import functools

import jax
import jax.numpy as jnp
from jax import lax
from jax.experimental import pallas as pl
from jax.experimental.pallas import tpu as pltpu
from jax.experimental.pallas import tpu_sc as plsc

SEQ_LEN = 4096
EMB = 1024
LANES = 16
CHUNK = 32


def kernel(input, encoding):
    mesh = plsc.VectorSubcoreMesh(core_axis_name="c", subcore_axis_name="s")
    info = plsc.get_sparse_core_info()
    num_workers = info.num_cores * info.num_subcores
    rows_per_worker = SEQ_LEN // num_workers
    nchunks = rows_per_worker // CHUNK

    @functools.partial(
        pl.kernel,
        mesh=mesh,
        out_type=jax.ShapeDtypeStruct((SEQ_LEN, EMB), jnp.float32),
        compiler_params=pltpu.CompilerParams(
            skip_device_barrier=True,
            disable_bounds_checks=True,
            disable_semaphore_checks=True,
        ),
        scratch_types=[
            pltpu.VMEM((LANES,), jnp.int32),
            pltpu.VMEM((2, CHUNK, EMB), jnp.float32),
            pltpu.SemaphoreType.DMA,
            pltpu.SemaphoreType.DMA,
            pltpu.SemaphoreType.DMA,
            pltpu.SemaphoreType.DMA,
        ],
    )
    def body(inp_hbm, enc_hbm, out_hbm, inp_v, buf, gs0, gs1, ss0, ss1):
        wid = lax.axis_index("s") * info.num_cores + lax.axis_index("c")
        pltpu.sync_copy(inp_hbm, inp_v.at[pl.ds(0, 2)])
        seq_from_input = inp_v[...][1]
        start = seq_from_input - SEQ_LEN
        base = wid * rows_per_worker
        src0 = pl.multiple_of(start + base, 8)
        dst0 = pl.multiple_of(base, 8)
        gsem = (gs0, gs1)
        ssem = (ss0, ss1)

        pend_g = [None, None]
        pend_s = [None, None]
        for i in range(nchunks):
            b = i & 1
            if pend_s[b] is not None:
                pend_s[b].wait()
            pend_g[b] = pltpu.async_copy(
                enc_hbm.at[pl.ds(src0 + i * CHUNK, CHUNK)], buf.at[b], gsem[b]
            )
            if i >= 1:
                ob = (i - 1) & 1
                pend_g[ob].wait()
                pend_s[ob] = pltpu.async_copy(
                    buf.at[ob], out_hbm.at[pl.ds(dst0 + (i - 1) * CHUNK, CHUNK)],
                    ssem[ob],
                )
        lb = (nchunks - 1) & 1
        pend_g[lb].wait()
        pend_s[lb] = pltpu.async_copy(
            buf.at[lb], out_hbm.at[pl.ds(dst0 + (nchunks - 1) * CHUNK, CHUNK)],
            ssem[lb],
        )
        pend_s[1 - lb].wait()
        pend_s[lb].wait()

    return body(input, encoding)
